# baseline (speedup 1.0000x reference)
import jax
import jax.numpy as jnp
from jax import lax
from jax.experimental import pallas as pl
from jax.experimental.pallas import tpu as pltpu
from jax.experimental.pallas import tpu_sc as plsc

N_NODES = 10000
N_EDGES = 320000
D = 128
DH = D // 2
NC = 2
NS = 16
LANES = 16

CHUNK = 80
N_CHUNKS = 252
NBUF = 4
EDGES_PER_TILE = N_CHUNKS * CHUNK
E_PAD = NS * EDGES_PER_TILE
ROWS_PER_TILE = 632
N_PAD = ROWS_PER_TILE * NS


def _spmm_body(h_hbm, src_hbm, dst_hbm, w_hbm, binit_hbm, out_hbm,
               src_v, dst_v, w_v, binit_v, rows, sem_i, sg, ss, acc_shared):
    sub = lax.axis_index("s")

    core = lax.axis_index("c")
    pltpu.async_copy(src_hbm.at[core, sub], src_v, sem_i)
    pltpu.async_copy(dst_hbm.at[sub], dst_v, sem_i)
    pltpu.async_copy(w_hbm.at[sub], w_v, sem_i)
    pltpu.sync_copy(binit_hbm, binit_v)

    bv = [binit_v[pl.ds(core * DH + k * LANES, LANES)]
          for k in range(DH // LANES)]

    def _fill(r, _):
        for kk in range(DH // LANES):
            rows[0][r, pl.ds(kk * LANES, LANES)] = bv[kk]
        return ()

    lax.fori_loop(0, CHUNK, _fill, (), unroll=4)
    r0 = sub * ROWS_PER_TILE
    for i in range(ROWS_PER_TILE // CHUNK):
        pltpu.sync_copy(rows[0], acc_shared.at[pl.ds(r0 + i * CHUNK, CHUNK)])
    rem = ROWS_PER_TILE % CHUNK
    pltpu.sync_copy(rows[0].at[pl.ds(0, rem)],
                    acc_shared.at[pl.ds(r0 + (ROWS_PER_TILE // CHUNK) * CHUNK, rem)])
    pltpu.make_async_copy(src_hbm.at[core, sub], src_v, sem_i).wait()
    pltpu.make_async_copy(dst_hbm.at[sub], dst_v, sem_i).wait()
    pltpu.make_async_copy(w_hbm.at[sub], w_v, sem_i).wait()
    plsc.subcore_barrier()

    def _start_gather(j, b):
        pltpu.async_copy(h_hbm.at[src_v.at[j]], rows[b], sg[b])

    def _wait_gather(j, b):
        pltpu.make_async_copy(h_hbm.at[src_v.at[j]], rows[b], sg[b]).wait()

    def _start_scatter(j, b):
        pltpu.async_copy(rows[b], acc_shared.at[dst_v.at[j]], ss[b], add=True)

    def _wait_scatter(j, b):
        pltpu.make_async_copy(rows[b], acc_shared.at[dst_v.at[j]], ss[b]).wait()

    def _scale(j, b):
        def _grp(g, _):
            wv = w_v[j, pl.ds(g * LANES, LANES)]
            for e in range(LANES):
                jj = g * LANES + e
                wb = jnp.take_along_axis(wv, jnp.full((LANES,), e, jnp.int32),
                                         axis=0)
                for k in range(DH // LANES):
                    sl = pl.ds(k * LANES, LANES)
                    rows[b][jj, sl] = rows[b][jj, sl] * wb
            return ()

        lax.fori_loop(0, CHUNK // LANES, _grp, ())

    _start_gather(0, 0)
    _start_gather(1, 1)

    def _group(g, _):
        for b in range(NBUF):
            j = NBUF * g + b
            _wait_gather(j, b)
            if b < 2:
                @pl.when(g > 0)
                def _free():
                    _wait_scatter(j - 2, (b + 2) % NBUF)

                _start_gather(j + 2, (b + 2) % NBUF)
            else:
                _wait_scatter(j - 2, (b + 2) % NBUF)

                @pl.when(g < N_CHUNKS // NBUF - 1)
                def _ahead():
                    _start_gather(j + 2, (b + 2) % NBUF)

            _scale(j, b)
            _start_scatter(j, b)
        return ()

    lax.fori_loop(0, N_CHUNKS // NBUF, _group, ())
    _wait_scatter(N_CHUNKS - 2, (N_CHUNKS - 2) % NBUF)
    _wait_scatter(N_CHUNKS - 1, (N_CHUNKS - 1) % NBUF)
    plsc.subcore_barrier()

    pltpu.sync_copy(acc_shared.at[pl.ds(r0, ROWS_PER_TILE)],
                    out_hbm.at[core, pl.ds(r0, ROWS_PER_TILE)])


def _spmm(h2, src, dst, w, binit):
    mesh = plsc.VectorSubcoreMesh(core_axis_name="c", subcore_axis_name="s",
                                  num_cores=NC, num_subcores=NS)
    return pl.kernel(
        _spmm_body,
        out_type=jax.ShapeDtypeStruct((NC, N_PAD, DH), jnp.float32),
        mesh=mesh,
        scratch_types=[
            pltpu.VMEM((N_CHUNKS, CHUNK), jnp.int32),
            pltpu.VMEM((N_CHUNKS, CHUNK), jnp.int32),
            pltpu.VMEM((N_CHUNKS, CHUNK), jnp.float32),
            pltpu.VMEM((D,), jnp.float32),
            [pltpu.VMEM((CHUNK, DH), jnp.float32) for _ in range(NBUF)],
            pltpu.SemaphoreType.DMA,
            [pltpu.SemaphoreType.DMA for _ in range(NBUF)],
            [pltpu.SemaphoreType.DMA for _ in range(NBUF)],
            pltpu.VMEM_SHARED((N_PAD, DH), jnp.float32),
        ],
        compiler_params=pltpu.CompilerParams(use_tc_tiling_on_sc=False),
    )(h2, src, dst, w, binit)


ROW_BLK = 1000


def _fuse_matmul_body(x_ref, h1_ref, w0_ref, w1_ref, z_ref):
    z_ref[...] = (jnp.dot(x_ref[...], w0_ref[...], preferred_element_type=jnp.float32)
                  + jnp.dot(h1_ref[0], w1_ref[:DH], preferred_element_type=jnp.float32)
                  + jnp.dot(h1_ref[1], w1_ref[DH:], preferred_element_type=jnp.float32))


def _fuse_matmul(x, h1_parts, w0, w1):
    grid = (N_NODES // ROW_BLK,)
    return pl.pallas_call(
        _fuse_matmul_body,
        grid=grid,
        in_specs=[
            pl.BlockSpec((ROW_BLK, D), lambda i: (i, 0)),
            pl.BlockSpec((NC, ROW_BLK, DH), lambda i: (0, i, 0)),
            pl.BlockSpec((D, D), lambda i: (0, 0)),
            pl.BlockSpec((D, D), lambda i: (0, 0)),
        ],
        out_specs=pl.BlockSpec((ROW_BLK, D), lambda i: (i, 0)),
        out_shape=jax.ShapeDtypeStruct((N_NODES, D), jnp.float32),
    )(x, h1_parts, w0, w1)


def _concat_body(parts_ref, out_ref):
    out_ref[...] = jnp.concatenate([parts_ref[0], parts_ref[1]], axis=1)


def _concat(parts):
    grid = (N_NODES // ROW_BLK,)
    return pl.pallas_call(
        _concat_body,
        grid=grid,
        in_specs=[pl.BlockSpec((NC, ROW_BLK, DH), lambda i: (0, i, 0))],
        out_specs=pl.BlockSpec((ROW_BLK, D), lambda i: (i, 0)),
        out_shape=jax.ShapeDtypeStruct((N_NODES, D), jnp.float32),
    )(parts)


def kernel(x, edge_index, edge_weight, W0, W1, b):
    pad = E_PAD - N_EDGES
    eshape = (NS, N_CHUNKS, CHUNK)
    dst = jnp.pad(edge_index[0].astype(jnp.int32), (0, pad)).reshape(eshape)
    src2 = 2 * jnp.pad(edge_index[1].astype(jnp.int32), (0, pad))
    src = jnp.stack([src2, src2 + 1]).reshape((NC,) + eshape)
    w = jnp.pad(edge_weight.astype(jnp.float32), (0, pad)).reshape(eshape)
    zeros_row = jnp.zeros((D,), jnp.float32)
    h1_parts = _spmm(x.reshape(2 * N_NODES, DH), src, dst, w, zeros_row)
    z = _fuse_matmul(x, h1_parts, W0, W1)
    out_parts = _spmm(z.reshape(2 * N_NODES, DH), src, dst, w, b)
    return _concat(out_parts)


# speedup vs baseline: 1.0761x; 1.0761x over previous
import jax
import jax.numpy as jnp
from jax import lax
from jax.experimental import pallas as pl
from jax.experimental.pallas import tpu as pltpu
from jax.experimental.pallas import tpu_sc as plsc

N_NODES = 10000
N_EDGES = 320000
D = 128
DH = D // 2
NC = 2
NS = 16
LANES = 16

CHUNK = 80
N_CHUNKS = 252
NBUF = 4
EDGES_PER_TILE = N_CHUNKS * CHUNK
E_PAD = NS * EDGES_PER_TILE
ROWS_PER_TILE = 632
N_PAD = ROWS_PER_TILE * NS


def _spmm_body(h_hbm, src_hbm, dst_hbm, w_hbm, out_hbm,
               src_v, dst_v, w_v, rows, sem_i, sg, ss, acc_shared):
    core = lax.axis_index("c")
    sub = lax.axis_index("s")

    pltpu.async_copy(src_hbm.at[sub], src_v, sem_i)
    pltpu.async_copy(dst_hbm.at[sub], dst_v, sem_i)
    pltpu.async_copy(w_hbm.at[sub], w_v, sem_i)

    zv = jnp.zeros((LANES,), jnp.float32)

    def _zero(j, _):
        r = j // (DH // LANES)
        k = j % (DH // LANES)
        rows[0][r, pl.ds(k * LANES, LANES)] = zv
        return ()

    lax.fori_loop(0, CHUNK * (DH // LANES), _zero, (), unroll=8)
    r0 = sub * ROWS_PER_TILE
    for i in range(ROWS_PER_TILE // CHUNK):
        pltpu.sync_copy(rows[0], acc_shared.at[pl.ds(r0 + i * CHUNK, CHUNK)])
    rem = ROWS_PER_TILE % CHUNK
    pltpu.sync_copy(rows[0].at[pl.ds(0, rem)],
                    acc_shared.at[pl.ds(r0 + (ROWS_PER_TILE // CHUNK) * CHUNK, rem)])
    pltpu.make_async_copy(src_hbm.at[sub], src_v, sem_i).wait()
    pltpu.make_async_copy(dst_hbm.at[sub], dst_v, sem_i).wait()
    pltpu.make_async_copy(w_hbm.at[sub], w_v, sem_i).wait()
    plsc.subcore_barrier()

    def _start_gather(j, b):
        pltpu.async_copy(h_hbm.at[core].at[src_v.at[j]], rows[b], sg[b])

    def _wait_gather(j, b):
        pltpu.make_async_copy(h_hbm.at[core].at[src_v.at[j]], rows[b], sg[b]).wait()

    def _start_scatter(j, b):
        pltpu.async_copy(rows[b], acc_shared.at[dst_v.at[j]], ss[b], add=True)

    def _wait_scatter(j, b):
        pltpu.make_async_copy(rows[b], acc_shared.at[dst_v.at[j]], ss[b]).wait()

    def _scale(j, b):
        def _grp(g, _):
            wv = w_v[j, pl.ds(g * LANES, LANES)]
            for e in range(LANES):
                jj = g * LANES + e
                we = wv[e]
                for k in range(DH // LANES):
                    sl = pl.ds(k * LANES, LANES)
                    rows[b][jj, sl] = rows[b][jj, sl] * we
            return ()

        lax.fori_loop(0, CHUNK // LANES, _grp, ())

    _start_gather(0, 0)
    _start_gather(1, 1)

    def _group(g, _):
        for b in range(NBUF):
            j = NBUF * g + b
            _wait_gather(j, b)
            if b < 2:
                @pl.when(g > 0)
                def _free():
                    _wait_scatter(j - 2, (b + 2) % NBUF)

                _start_gather(j + 2, (b + 2) % NBUF)
            else:
                _wait_scatter(j - 2, (b + 2) % NBUF)

                @pl.when(g < N_CHUNKS // NBUF - 1)
                def _ahead():
                    _start_gather(j + 2, (b + 2) % NBUF)

            _scale(j, b)
            _start_scatter(j, b)
        return ()

    lax.fori_loop(0, N_CHUNKS // NBUF, _group, ())
    _wait_scatter(N_CHUNKS - 2, (N_CHUNKS - 2) % NBUF)
    _wait_scatter(N_CHUNKS - 1, (N_CHUNKS - 1) % NBUF)
    plsc.subcore_barrier()

    pltpu.sync_copy(acc_shared.at[pl.ds(r0, ROWS_PER_TILE)],
                    out_hbm.at[core, pl.ds(r0, ROWS_PER_TILE)])


def _spmm_split(h_split, src, dst, w):
    mesh = plsc.VectorSubcoreMesh(core_axis_name="c", subcore_axis_name="s",
                                  num_cores=NC, num_subcores=NS)
    return pl.kernel(
        _spmm_body,
        out_type=jax.ShapeDtypeStruct((NC, N_PAD, DH), jnp.float32),
        mesh=mesh,
        scratch_types=[
            pltpu.VMEM((N_CHUNKS, CHUNK), jnp.int32),
            pltpu.VMEM((N_CHUNKS, CHUNK), jnp.int32),
            pltpu.VMEM((N_CHUNKS, CHUNK), jnp.float32),
            [pltpu.VMEM((CHUNK, DH), jnp.float32) for _ in range(NBUF)],
            pltpu.SemaphoreType.DMA,
            [pltpu.SemaphoreType.DMA for _ in range(NBUF)],
            [pltpu.SemaphoreType.DMA for _ in range(NBUF)],
            pltpu.VMEM_SHARED((N_PAD, DH), jnp.float32),
        ],
        compiler_params=pltpu.CompilerParams(use_tc_tiling_on_sc=False),
    )(h_split, src, dst, w)


ROW_BLK = 1000


def _split_body(x_ref, out_ref):
    out_ref[0] = x_ref[:, :DH]
    out_ref[1] = x_ref[:, DH:]


def _split(x):
    grid = (N_NODES // ROW_BLK,)
    return pl.pallas_call(
        _split_body,
        grid=grid,
        in_specs=[pl.BlockSpec((ROW_BLK, D), lambda i: (i, 0))],
        out_specs=pl.BlockSpec((NC, ROW_BLK, DH), lambda i: (0, i, 0)),
        out_shape=jax.ShapeDtypeStruct((NC, N_PAD, DH), jnp.float32),
    )(x)


def _fuse_matmul_body(x_ref, parts_ref, w0_ref, w1_ref, z_ref):
    h1 = jnp.concatenate([parts_ref[0], parts_ref[1]], axis=1)
    z = (jnp.dot(x_ref[...], w0_ref[...], preferred_element_type=jnp.float32)
         + jnp.dot(h1, w1_ref[...], preferred_element_type=jnp.float32))
    z_ref[0] = z[:, :DH]
    z_ref[1] = z[:, DH:]


def _fuse_matmul(x, parts, w0, w1):
    grid = (N_NODES // ROW_BLK,)
    return pl.pallas_call(
        _fuse_matmul_body,
        grid=grid,
        in_specs=[
            pl.BlockSpec((ROW_BLK, D), lambda i: (i, 0)),
            pl.BlockSpec((NC, ROW_BLK, DH), lambda i: (0, i, 0)),
            pl.BlockSpec((D, D), lambda i: (0, 0)),
            pl.BlockSpec((D, D), lambda i: (0, 0)),
        ],
        out_specs=pl.BlockSpec((NC, ROW_BLK, DH), lambda i: (0, i, 0)),
        out_shape=jax.ShapeDtypeStruct((NC, N_PAD, DH), jnp.float32),
    )(x, parts, w0, w1)


def _combine_bias_body(parts_ref, b_ref, out_ref):
    out_ref[...] = (jnp.concatenate([parts_ref[0], parts_ref[1]], axis=1)
                    + b_ref[...])


def _combine_bias(parts, b):
    grid = (N_NODES // ROW_BLK,)
    return pl.pallas_call(
        _combine_bias_body,
        grid=grid,
        in_specs=[
            pl.BlockSpec((NC, ROW_BLK, DH), lambda i: (0, i, 0)),
            pl.BlockSpec((1, D), lambda i: (0, 0)),
        ],
        out_specs=pl.BlockSpec((ROW_BLK, D), lambda i: (i, 0)),
        out_shape=jax.ShapeDtypeStruct((N_NODES, D), jnp.float32),
    )(parts, b)


def kernel(x, edge_index, edge_weight, W0, W1, b):
    pad = E_PAD - N_EDGES
    eshape = (NS, N_CHUNKS, CHUNK)
    dst = jnp.pad(edge_index[0].astype(jnp.int32), (0, pad)).reshape(eshape)
    src = jnp.pad(edge_index[1].astype(jnp.int32), (0, pad)).reshape(eshape)
    w = jnp.pad(edge_weight.astype(jnp.float32), (0, pad)).reshape(eshape)
    x_split = _split(x)
    h1_parts = _spmm_split(x_split, src, dst, w)
    z_split = _fuse_matmul(x, h1_parts, W0, W1)
    out_parts = _spmm_split(z_split, src, dst, w)
    return _combine_bias(out_parts, b.reshape(1, D))


# speedup vs baseline: 1.8986x; 1.7643x over previous
import jax
import jax.numpy as jnp
from jax import lax
from jax.experimental import pallas as pl
from jax.experimental.pallas import tpu as pltpu
from jax.experimental.pallas import tpu_sc as plsc

N_NODES = 10000
N_EDGES = 320000
D = 128
DH = D // 2
NC = 2
NS = 16
LANES = 16

CHUNK = 80
N_CHUNKS = 252
NBUF = 4
EDGES_PER_TILE = N_CHUNKS * CHUNK
E_PAD = NS * EDGES_PER_TILE
ROWS_PER_TILE = 632
N_PAD = ROWS_PER_TILE * NS


def _spmm_body(h_hbm, src_hbm, dst_hbm, w_hbm, out_hbm,
               src_v, dst_v, w_v, rows, sem_i, sg, ss, acc_shared):
    core = lax.axis_index("c")
    sub = lax.axis_index("s")

    pltpu.async_copy(src_hbm.at[sub], src_v, sem_i)
    pltpu.async_copy(dst_hbm.at[sub], dst_v, sem_i)
    pltpu.async_copy(w_hbm.at[sub], w_v, sem_i)

    zv = jnp.zeros((LANES,), jnp.float32)

    def _zero(j, _):
        r = j // (DH // LANES)
        k = j % (DH // LANES)
        rows[0][r, pl.ds(k * LANES, LANES)] = zv
        return ()

    lax.fori_loop(0, CHUNK * (DH // LANES), _zero, (), unroll=8)
    r0 = sub * ROWS_PER_TILE
    for i in range(ROWS_PER_TILE // CHUNK):
        pltpu.sync_copy(rows[0], acc_shared.at[pl.ds(r0 + i * CHUNK, CHUNK)])
    rem = ROWS_PER_TILE % CHUNK
    pltpu.sync_copy(rows[0].at[pl.ds(0, rem)],
                    acc_shared.at[pl.ds(r0 + (ROWS_PER_TILE // CHUNK) * CHUNK, rem)])
    pltpu.make_async_copy(src_hbm.at[sub], src_v, sem_i).wait()
    pltpu.make_async_copy(dst_hbm.at[sub], dst_v, sem_i).wait()
    pltpu.make_async_copy(w_hbm.at[sub], w_v, sem_i).wait()
    plsc.subcore_barrier()

    def _start_gather(j, b):
        pltpu.async_copy(h_hbm.at[core].at[src_v.at[j]], rows[b], sg[b])

    def _wait_gather(j, b):
        pltpu.make_async_copy(h_hbm.at[core].at[src_v.at[j]], rows[b], sg[b]).wait()

    def _start_scatter(j, b):
        pltpu.async_copy(rows[b], acc_shared.at[dst_v.at[j]], ss[b], add=True)

    def _wait_scatter(j, b):
        pltpu.make_async_copy(rows[b], acc_shared.at[dst_v.at[j]], ss[b]).wait()

    def _scale(j, b):
        def _grp(g, _):
            wv = w_v[j, pl.ds(g * LANES, LANES)]
            for e in range(LANES):
                jj = g * LANES + e
                we = wv[e]
                for k in range(DH // LANES):
                    sl = pl.ds(k * LANES, LANES)
                    rows[b][jj, sl] = rows[b][jj, sl] * we
            return ()

        lax.fori_loop(0, CHUNK // LANES, _grp, (), unroll=True)

    _start_gather(0, 0)
    _start_gather(1, 1)

    def _group(g, _):
        for b in range(NBUF):
            j = NBUF * g + b
            _wait_gather(j, b)
            if b < 2:
                @pl.when(g > 0)
                def _free():
                    _wait_scatter(j - 2, (b + 2) % NBUF)

                _start_gather(j + 2, (b + 2) % NBUF)
            else:
                _wait_scatter(j - 2, (b + 2) % NBUF)

                @pl.when(g < N_CHUNKS // NBUF - 1)
                def _ahead():
                    _start_gather(j + 2, (b + 2) % NBUF)

            _scale(j, b)
            _start_scatter(j, b)
        return ()

    lax.fori_loop(0, N_CHUNKS // NBUF, _group, ())
    _wait_scatter(N_CHUNKS - 2, (N_CHUNKS - 2) % NBUF)
    _wait_scatter(N_CHUNKS - 1, (N_CHUNKS - 1) % NBUF)
    plsc.subcore_barrier()

    pltpu.sync_copy(acc_shared.at[pl.ds(r0, ROWS_PER_TILE)],
                    out_hbm.at[core, pl.ds(r0, ROWS_PER_TILE)])


def _spmm_split(h_split, src, dst, w):
    mesh = plsc.VectorSubcoreMesh(core_axis_name="c", subcore_axis_name="s",
                                  num_cores=NC, num_subcores=NS)
    return pl.kernel(
        _spmm_body,
        out_type=jax.ShapeDtypeStruct((NC, N_PAD, DH), jnp.float32),
        mesh=mesh,
        scratch_types=[
            pltpu.VMEM((N_CHUNKS, CHUNK), jnp.int32),
            pltpu.VMEM((N_CHUNKS, CHUNK), jnp.int32),
            pltpu.VMEM((N_CHUNKS, CHUNK), jnp.float32),
            [pltpu.VMEM((CHUNK, DH), jnp.float32) for _ in range(NBUF)],
            pltpu.SemaphoreType.DMA,
            [pltpu.SemaphoreType.DMA for _ in range(NBUF)],
            [pltpu.SemaphoreType.DMA for _ in range(NBUF)],
            pltpu.VMEM_SHARED((N_PAD, DH), jnp.float32),
        ],
        compiler_params=pltpu.CompilerParams(use_tc_tiling_on_sc=False),
    )(h_split, src, dst, w)


ROW_BLK = 1000


def _split_body(x_ref, out_ref):
    out_ref[0] = x_ref[:, :DH]
    out_ref[1] = x_ref[:, DH:]


def _split(x):
    grid = (N_NODES // ROW_BLK,)
    return pl.pallas_call(
        _split_body,
        grid=grid,
        in_specs=[pl.BlockSpec((ROW_BLK, D), lambda i: (i, 0))],
        out_specs=pl.BlockSpec((NC, ROW_BLK, DH), lambda i: (0, i, 0)),
        out_shape=jax.ShapeDtypeStruct((NC, N_PAD, DH), jnp.float32),
    )(x)


def _fuse_matmul_body(x_ref, parts_ref, w0_ref, w1_ref, z_ref):
    h1 = jnp.concatenate([parts_ref[0], parts_ref[1]], axis=1)
    z = (jnp.dot(x_ref[...], w0_ref[...], preferred_element_type=jnp.float32)
         + jnp.dot(h1, w1_ref[...], preferred_element_type=jnp.float32))
    z_ref[0] = z[:, :DH]
    z_ref[1] = z[:, DH:]


def _fuse_matmul(x, parts, w0, w1):
    grid = (N_NODES // ROW_BLK,)
    return pl.pallas_call(
        _fuse_matmul_body,
        grid=grid,
        in_specs=[
            pl.BlockSpec((ROW_BLK, D), lambda i: (i, 0)),
            pl.BlockSpec((NC, ROW_BLK, DH), lambda i: (0, i, 0)),
            pl.BlockSpec((D, D), lambda i: (0, 0)),
            pl.BlockSpec((D, D), lambda i: (0, 0)),
        ],
        out_specs=pl.BlockSpec((NC, ROW_BLK, DH), lambda i: (0, i, 0)),
        out_shape=jax.ShapeDtypeStruct((NC, N_PAD, DH), jnp.float32),
    )(x, parts, w0, w1)


def _combine_bias_body(parts_ref, b_ref, out_ref):
    out_ref[...] = (jnp.concatenate([parts_ref[0], parts_ref[1]], axis=1)
                    + b_ref[...])


def _combine_bias(parts, b):
    grid = (N_NODES // ROW_BLK,)
    return pl.pallas_call(
        _combine_bias_body,
        grid=grid,
        in_specs=[
            pl.BlockSpec((NC, ROW_BLK, DH), lambda i: (0, i, 0)),
            pl.BlockSpec((1, D), lambda i: (0, 0)),
        ],
        out_specs=pl.BlockSpec((ROW_BLK, D), lambda i: (i, 0)),
        out_shape=jax.ShapeDtypeStruct((N_NODES, D), jnp.float32),
    )(parts, b)


def kernel(x, edge_index, edge_weight, W0, W1, b):
    pad = E_PAD - N_EDGES
    eshape = (NS, N_CHUNKS, CHUNK)
    dst = jnp.pad(edge_index[0].astype(jnp.int32), (0, pad)).reshape(eshape)
    src = jnp.pad(edge_index[1].astype(jnp.int32), (0, pad)).reshape(eshape)
    w = jnp.pad(edge_weight.astype(jnp.float32), (0, pad)).reshape(eshape)
    x_split = _split(x)
    h1_parts = _spmm_split(x_split, src, dst, w)
    z_split = _fuse_matmul(x, h1_parts, W0, W1)
    out_parts = _spmm_split(z_split, src, dst, w)
    return _combine_bias(out_parts, b.reshape(1, D))
